# Initial kernel scaffold; baseline (speedup 1.0000x reference)
import jax
import jax.numpy as jnp
from jax.experimental import pallas as pl


def kernel(x, edge_index, edge_attr, batch, W, b):
    raise NotImplementedError("write your pallas kernel here")


# speedup vs baseline: 115.7864x; 115.7864x over previous
import functools

import jax
import jax.numpy as jnp
from jax import lax
from jax.experimental import pallas as pl
from jax.experimental.pallas import tpu as pltpu
from jax.experimental.pallas import tpu_sc as plsc

L = 16
NC = 2
NS = 16
NW = NC * NS
NG = 4
EMBED = 8


def _deg_partials_fn(e_pad, n):
    ept = e_pad // NW
    groups = ept // L
    mesh = plsc.VectorSubcoreMesh(core_axis_name="c", subcore_axis_name="s")

    @functools.partial(
        pl.kernel,
        mesh=mesh,
        out_type=jax.ShapeDtypeStruct((NW, n), jnp.float32),
        scratch_types=[
            pltpu.VMEM((ept,), jnp.int32),
            pltpu.VMEM((ept,), jnp.float32),
            pltpu.VMEM((n,), jnp.float32),
            pltpu.SemaphoreType.DMA,
        ],
        compiler_params=pltpu.CompilerParams(needs_layout_passes=False),
    )
    def k(dst_hbm, w_hbm, out_hbm, dst_v, w_v, acc, sem):
        tid = lax.axis_index("s") * NC + lax.axis_index("c")
        base = tid * ept
        cp1 = pltpu.async_copy(dst_hbm.at[pl.ds(base, ept)], dst_v, sem)
        cp2 = pltpu.async_copy(w_hbm.at[pl.ds(base, ept)], w_v, sem)

        def zero(i, _):
            acc[pl.ds(i * L, L)] = jnp.zeros((L,), jnp.float32)
            return 0

        lax.fori_loop(0, n // L, zero, 0)
        cp1.wait()
        cp2.wait()

        def body(i, _):
            d = dst_v[pl.ds(i * L, L)]
            wv = w_v[pl.ds(i * L, L)]
            plsc.addupdate_scatter(acc, [d], wv)
            return 0

        lax.fori_loop(0, groups, body, 0)
        pltpu.sync_copy(acc, out_hbm.at[tid])

    return k


def _edge_partials_fn(e_pad, n):
    ept = e_pad // NW
    groups = ept // L
    acc_len = NG * n
    mesh = plsc.VectorSubcoreMesh(core_axis_name="c", subcore_axis_name="s")

    @functools.partial(
        pl.kernel,
        mesh=mesh,
        out_type=jax.ShapeDtypeStruct((NW, acc_len), jnp.float32),
        scratch_types=[
            pltpu.VMEM((ept,), jnp.int32),
            pltpu.VMEM((ept,), jnp.int32),
            pltpu.VMEM((ept,), jnp.float32),
            pltpu.VMEM((n,), jnp.float32),
            pltpu.VMEM((acc_len,), jnp.float32),
            pltpu.VMEM((acc_len,), jnp.float32),
            pltpu.SemaphoreType.DMA,
        ],
        compiler_params=pltpu.CompilerParams(needs_layout_passes=False),
    )
    def k(src_hbm, dst_hbm, w_hbm, dinv_hbm, hs_hbm, out_hbm,
          src_v, dst_v, w_v, dinv_v, hs_v, acc, sem):
        tid = lax.axis_index("s") * NC + lax.axis_index("c")
        base = tid * ept
        cp1 = pltpu.async_copy(src_hbm.at[pl.ds(base, ept)], src_v, sem)
        cp2 = pltpu.async_copy(dst_hbm.at[pl.ds(base, ept)], dst_v, sem)
        cp3 = pltpu.async_copy(w_hbm.at[pl.ds(base, ept)], w_v, sem)
        cp4 = pltpu.async_copy(dinv_hbm, dinv_v, sem)
        cp5 = pltpu.async_copy(hs_hbm, hs_v, sem)

        def zero(i, _):
            acc[pl.ds(i * L, L)] = jnp.zeros((L,), jnp.float32)
            return 0

        lax.fori_loop(0, acc_len // L, zero, 0)
        cp1.wait()
        cp2.wait()
        cp3.wait()
        cp4.wait()
        cp5.wait()

        def body(i, _):
            s = src_v[pl.ds(i * L, L)]
            d = dst_v[pl.ds(i * L, L)]
            wv = w_v[pl.ds(i * L, L)]
            di_s = plsc.load_gather(dinv_v, [s])
            di_d = plsc.load_gather(dinv_v, [d])
            val = wv * di_s * di_d
            for g in range(NG):
                off = jnp.int32(g * n)
                hv = plsc.load_gather(hs_v, [s + off])
                plsc.addupdate_scatter(acc, [d + off], val * hv)
            return 0

        lax.fori_loop(0, groups, body, 0)
        pltpu.sync_copy(acc, out_hbm.at[tid])

    return k


def _prologue_tc(degp, x, wc):
    n = x.shape[0]

    def body(degp_ref, x_ref, wc_ref, dinv_ref, hs_ref):
        deg = 1.0 + jnp.sum(degp_ref[...], axis=0, keepdims=True)
        dinv_ref[...] = jnp.where(deg > 0, lax.rsqrt(deg), 0.0)
        hs_ref[...] = lax.dot_general(
            wc_ref[...], x_ref[...], (((1,), (1,)), ((), ())),
            preferred_element_type=jnp.float32)

    return pl.pallas_call(
        body,
        out_shape=[
            jax.ShapeDtypeStruct((1, n), jnp.float32),
            jax.ShapeDtypeStruct((NG, n), jnp.float32),
        ],
    )(degp, x, wc)


def _epilogue_tc(outp, hs, dinv, x0, b0):
    n = hs.shape[1]

    def body(outp_ref, hs_ref, dinv_ref, x0_ref, b0_ref, o_ref):
        p = jnp.sum(outp_ref[...], axis=0)
        di = dinv_ref[...]
        o_ref[...] = p + di * di * hs_ref[...] + b0_ref[...] - x0_ref[...]

    return pl.pallas_call(
        body,
        out_shape=jax.ShapeDtypeStruct((NG, n), jnp.float32),
    )(outp, hs, dinv, x0, b0)


def kernel(x, edge_index, edge_attr, batch, W, b):
    n = x.shape[0]
    e = edge_attr.shape[0]
    src = edge_index[0].astype(jnp.int32)
    dst = edge_index[1].astype(jnp.int32)
    w = edge_attr.astype(jnp.float32)

    chunk = NW * L
    e_pad = ((e + chunk - 1) // chunk) * chunk
    if e_pad != e:
        pad = e_pad - e
        src = jnp.concatenate([src, jnp.zeros((pad,), jnp.int32)])
        dst = jnp.concatenate([dst, jnp.zeros((pad,), jnp.int32)])
        w = jnp.concatenate([w, jnp.zeros((pad,), jnp.float32)])

    wc = W[:, :, 0]
    b0 = b[:, 0].reshape(NG, 1)
    x0 = x[:, 0].reshape(1, n)

    degp = _deg_partials_fn(e_pad, n)(dst, w)
    dinv, hs = _prologue_tc(degp, x, wc)
    outp = _edge_partials_fn(e_pad, n)(
        src, dst, w, dinv.reshape(n), hs.reshape(NG * n))
    v = _epilogue_tc(outp.reshape(NW, NG, n), hs, dinv, x0, b0)
    return v.reshape(NG, n // EMBED, EMBED)


# speedup vs baseline: 134.9756x; 1.1657x over previous
import functools

import jax
import jax.numpy as jnp
from jax import lax
from jax.experimental import pallas as pl
from jax.experimental.pallas import tpu as pltpu
from jax.experimental.pallas import tpu_sc as plsc

L = 16
NC = 2
NS = 16
NW = NC * NS
NG = 4
EMBED = 8


def _deg_partials_fn(e_pad, n):
    ept = e_pad // NW
    groups = ept // L
    mesh = plsc.VectorSubcoreMesh(core_axis_name="c", subcore_axis_name="s")

    @functools.partial(
        pl.kernel,
        mesh=mesh,
        out_type=jax.ShapeDtypeStruct((NW, n), jnp.float32),
        scratch_types=[
            pltpu.VMEM((ept,), jnp.int32),
            pltpu.VMEM((ept,), jnp.float32),
            pltpu.VMEM((n,), jnp.float32),
            pltpu.SemaphoreType.DMA,
        ],
        compiler_params=pltpu.CompilerParams(needs_layout_passes=False),
    )
    def k(dst_hbm, w_hbm, out_hbm, dst_v, w_v, acc, sem):
        tid = lax.axis_index("s") * NC + lax.axis_index("c")
        base = tid * ept
        cp1 = pltpu.async_copy(dst_hbm.at[pl.ds(base, ept)], dst_v, sem)
        cp2 = pltpu.async_copy(w_hbm.at[pl.ds(base, ept)], w_v, sem)

        @plsc.parallel_loop(0, n, L, unroll=8)
        def _(i):
            acc[pl.ds(i, L)] = jnp.zeros((L,), jnp.float32)

        cp1.wait()
        cp2.wait()

        @plsc.parallel_loop(0, ept, L, unroll=4)
        def _(i):
            d = dst_v[pl.ds(i, L)]
            wv = w_v[pl.ds(i, L)]
            plsc.addupdate_scatter(acc, [d], wv)

        pltpu.sync_copy(acc, out_hbm.at[tid])

    return k


def _edge_partials_fn(e_pad, n):
    ept = e_pad // NW
    groups = ept // L
    acc_len = NG * n
    mesh = plsc.VectorSubcoreMesh(core_axis_name="c", subcore_axis_name="s")

    @functools.partial(
        pl.kernel,
        mesh=mesh,
        out_type=jax.ShapeDtypeStruct((NW, acc_len), jnp.float32),
        scratch_types=[
            pltpu.VMEM((ept,), jnp.int32),
            pltpu.VMEM((ept,), jnp.int32),
            pltpu.VMEM((ept,), jnp.float32),
            pltpu.VMEM((n,), jnp.float32),
            pltpu.VMEM((acc_len,), jnp.float32),
            pltpu.VMEM((acc_len,), jnp.float32),
            pltpu.SemaphoreType.DMA,
        ],
        compiler_params=pltpu.CompilerParams(needs_layout_passes=False),
    )
    def k(src_hbm, dst_hbm, w_hbm, dinv_hbm, hs_hbm, out_hbm,
          src_v, dst_v, w_v, dinv_v, hs_v, acc, sem):
        tid = lax.axis_index("s") * NC + lax.axis_index("c")
        base = tid * ept
        cp1 = pltpu.async_copy(src_hbm.at[pl.ds(base, ept)], src_v, sem)
        cp2 = pltpu.async_copy(dst_hbm.at[pl.ds(base, ept)], dst_v, sem)
        cp3 = pltpu.async_copy(w_hbm.at[pl.ds(base, ept)], w_v, sem)
        cp4 = pltpu.async_copy(dinv_hbm, dinv_v, sem)
        cp5 = pltpu.async_copy(hs_hbm, hs_v, sem)

        @plsc.parallel_loop(0, acc_len, L, unroll=8)
        def _(i):
            acc[pl.ds(i, L)] = jnp.zeros((L,), jnp.float32)

        cp1.wait()
        cp2.wait()
        cp3.wait()
        cp4.wait()
        cp5.wait()

        @plsc.parallel_loop(0, ept, L, unroll=2)
        def _(i):
            s = src_v[pl.ds(i, L)]
            d = dst_v[pl.ds(i, L)]
            wv = w_v[pl.ds(i, L)]
            di_s = plsc.load_gather(dinv_v, [s])
            di_d = plsc.load_gather(dinv_v, [d])
            val = wv * di_s * di_d
            for g in range(NG):
                off = jnp.int32(g * n)
                hv = plsc.load_gather(hs_v, [s + off])
                plsc.addupdate_scatter(acc, [d + off], val * hv)

        pltpu.sync_copy(acc, out_hbm.at[tid])

    return k


def _prologue_tc(degp, x, wc):
    n = x.shape[0]

    def body(degp_ref, x_ref, wc_ref, dinv_ref, hs_ref):
        deg = 1.0 + jnp.sum(degp_ref[...], axis=0, keepdims=True)
        dinv_ref[...] = jnp.where(deg > 0, lax.rsqrt(deg), 0.0)
        hs_ref[...] = lax.dot_general(
            wc_ref[...], x_ref[...], (((1,), (1,)), ((), ())),
            preferred_element_type=jnp.float32)

    return pl.pallas_call(
        body,
        out_shape=[
            jax.ShapeDtypeStruct((1, n), jnp.float32),
            jax.ShapeDtypeStruct((NG, n), jnp.float32),
        ],
    )(degp, x, wc)


def _epilogue_tc(outp, hs, dinv, x0, b0):
    n = hs.shape[1]

    def body(outp_ref, hs_ref, dinv_ref, x0_ref, b0_ref, o_ref):
        p = jnp.sum(outp_ref[...], axis=0)
        di = dinv_ref[...]
        o_ref[...] = p + di * di * hs_ref[...] + b0_ref[...] - x0_ref[...]

    return pl.pallas_call(
        body,
        out_shape=jax.ShapeDtypeStruct((NG, n), jnp.float32),
    )(outp, hs, dinv, x0, b0)


def kernel(x, edge_index, edge_attr, batch, W, b):
    n = x.shape[0]
    e = edge_attr.shape[0]
    src = edge_index[0].astype(jnp.int32)
    dst = edge_index[1].astype(jnp.int32)
    w = edge_attr.astype(jnp.float32)

    chunk = NW * L
    e_pad = ((e + chunk - 1) // chunk) * chunk
    if e_pad != e:
        pad = e_pad - e
        src = jnp.concatenate([src, jnp.zeros((pad,), jnp.int32)])
        dst = jnp.concatenate([dst, jnp.zeros((pad,), jnp.int32)])
        w = jnp.concatenate([w, jnp.zeros((pad,), jnp.float32)])

    wc = W[:, :, 0]
    b0 = b[:, 0].reshape(NG, 1)
    x0 = x[:, 0].reshape(1, n)

    degp = _deg_partials_fn(e_pad, n)(dst, w)
    dinv, hs = _prologue_tc(degp, x, wc)
    outp = _edge_partials_fn(e_pad, n)(
        src, dst, w, dinv.reshape(n), hs.reshape(NG * n))
    v = _epilogue_tc(outp.reshape(NW, NG, n), hs, dinv, x0, b0)
    return v.reshape(NG, n // EMBED, EMBED)


# speedup vs baseline: 142.8446x; 1.0583x over previous
import functools

import jax
import jax.numpy as jnp
from jax import lax
from jax.experimental import pallas as pl
from jax.experimental.pallas import tpu as pltpu
from jax.experimental.pallas import tpu_sc as plsc

L = 16
NC = 2
NS = 16
NW = NC * NS
NG = 4
GPC = NG // NC
TPG = NS // GPC
EMBED = 8
RSQRT_MAGIC = 0x5F3759DF


def _sc_mega_fn(e_pad, n):
    ept_d = e_pad // NS
    ept_m = e_pad // TPG
    half = ept_d // 2
    n_pad = -(-n // 2048) * 2048
    npt = n_pad // NS
    mesh = plsc.VectorSubcoreMesh(core_axis_name="c", subcore_axis_name="s")

    @functools.partial(
        pl.kernel,
        mesh=mesh,
        out_type=(
            jax.ShapeDtypeStruct((NW, n), jnp.float32),
            jax.ShapeDtypeStruct((NC, n_pad), jnp.float32),
        ),
        scratch_types=[
            pltpu.VMEM((half,), jnp.int32),
            pltpu.VMEM((half,), jnp.int32),
            pltpu.VMEM((half,), jnp.float32),
            pltpu.VMEM((n_pad,), jnp.float32),
            pltpu.VMEM((2 * npt,), jnp.float32),
            pltpu.VMEM((n,), jnp.float32),
            pltpu.VMEM((n,), jnp.float32),
            pltpu.VMEM_SHARED((NC, NS, n_pad), jnp.float32),
            pltpu.VMEM_SHARED((NC, n_pad), jnp.float32),
            pltpu.SemaphoreType.DMA,
            pltpu.SemaphoreType.DMA,
        ],
        compiler_params=pltpu.CompilerParams(needs_layout_passes=False),
    )
    def k(src_hbm, dst_hbm, w_hbm, hs_hbm, out_hbm, dinv_hbm,
          src_v, dst_v, w_v, dd, sl, hs_v, acc, shp, shd, sem_a, sem_b):
        cid = lax.axis_index("c")
        sid = lax.axis_index("s")
        tid = cid * NS + sid
        gene = cid * GPC + sid // TPG
        base_d = sid * ept_d
        base_m = (sid % TPG) * ept_m

        cp_d0 = pltpu.async_copy(
            dst_hbm.at[pl.ds(base_d, half)], dst_v, sem_a)
        cp_w0 = pltpu.async_copy(w_hbm.at[pl.ds(base_d, half)], w_v, sem_a)
        cp_h = pltpu.async_copy(hs_hbm.at[pl.ds(gene * n, n)], hs_v, sem_b)

        @plsc.parallel_loop(0, n_pad, L, unroll=8)
        def _(i):
            dd[pl.ds(i, L)] = jnp.zeros((L,), jnp.float32)

        @plsc.parallel_loop(0, n, L, unroll=8)
        def _(i):
            acc[pl.ds(i, L)] = jnp.zeros((L,), jnp.float32)

        def deg_round():
            @plsc.parallel_loop(0, half, L, unroll=4)
            def _(i):
                d = dst_v[pl.ds(i, L)]
                wv = w_v[pl.ds(i, L)]
                plsc.addupdate_scatter(dd, [d], wv)

        cp_d0.wait()
        cp_w0.wait()
        deg_round()
        cp_d1 = pltpu.async_copy(
            dst_hbm.at[pl.ds(base_d + half, half)], dst_v, sem_a)
        cp_w1 = pltpu.async_copy(
            w_hbm.at[pl.ds(base_d + half, half)], w_v, sem_a)
        cp_d1.wait()
        cp_w1.wait()
        deg_round()

        cp_s = pltpu.async_copy(src_hbm.at[pl.ds(base_m, half)], src_v, sem_a)
        cp_d = pltpu.async_copy(dst_hbm.at[pl.ds(base_m, half)], dst_v, sem_a)
        cp_w = pltpu.async_copy(w_hbm.at[pl.ds(base_m, half)], w_v, sem_a)

        pltpu.sync_copy(dd, shp.at[cid, sid])
        plsc.subcore_barrier()
        pltpu.sync_copy(shp.at[cid, 0, pl.ds(sid * npt, npt)],
                        sl.at[pl.ds(0, npt)])
        for t in range(1, NS):
            pltpu.sync_copy(shp.at[cid, t, pl.ds(sid * npt, npt)],
                            sl.at[pl.ds(npt, npt)])

            @plsc.parallel_loop(0, npt, L)
            def _(i):
                sl[pl.ds(i, L)] = sl[pl.ds(i, L)] + sl[pl.ds(npt + i, L)]

        @plsc.parallel_loop(0, npt, L, unroll=2)
        def _(i):
            deg = sl[pl.ds(i, L)] + 1.0
            ii = jnp.int32(RSQRT_MAGIC) - lax.shift_right_logical(
                plsc.bitcast(deg, jnp.int32), 1)
            y = plsc.bitcast(ii, jnp.float32)
            for _ in range(3):
                y = y * (1.5 - 0.5 * deg * y * y)
            sl[pl.ds(i, L)] = jnp.where(deg > 0, y, 0.0)

        pltpu.sync_copy(sl.at[pl.ds(0, npt)], shd.at[cid, pl.ds(sid * npt, npt)])
        plsc.subcore_barrier()
        pltpu.sync_copy(shd.at[cid], dd)

        @pl.when(sid == 0)
        def _():
            pltpu.sync_copy(dd, dinv_hbm.at[cid])

        cp_h.wait()
        cp_s.wait()
        cp_d.wait()
        cp_w.wait()

        def msg_round():
            @plsc.parallel_loop(0, half, L, unroll=2)
            def _(i):
                s = src_v[pl.ds(i, L)]
                d = dst_v[pl.ds(i, L)]
                wv = w_v[pl.ds(i, L)]
                di_s = plsc.load_gather(dd, [s])
                di_d = plsc.load_gather(dd, [d])
                hv = plsc.load_gather(hs_v, [s])
                plsc.addupdate_scatter(acc, [d], wv * di_s * di_d * hv)

        msg_round()
        for r in range(1, ept_m // half):
            cp_s = pltpu.async_copy(
                src_hbm.at[pl.ds(base_m + r * half, half)], src_v, sem_a)
            cp_d = pltpu.async_copy(
                dst_hbm.at[pl.ds(base_m + r * half, half)], dst_v, sem_a)
            cp_w = pltpu.async_copy(
                w_hbm.at[pl.ds(base_m + r * half, half)], w_v, sem_a)
            cp_s.wait()
            cp_d.wait()
            cp_w.wait()
            msg_round()

        pltpu.sync_copy(acc, out_hbm.at[tid])

    return k


def _hs_tc(x, wc):
    n = x.shape[0]

    def body(x_ref, wc_ref, hs_ref):
        hs_ref[...] = lax.dot_general(
            wc_ref[...], x_ref[...], (((1,), (1,)), ((), ())),
            preferred_element_type=jnp.float32)

    return pl.pallas_call(
        body,
        out_shape=jax.ShapeDtypeStruct((NG, n), jnp.float32),
    )(x, wc)


def _epilogue_tc(outp, hs, dinv, x0, b0):
    n = hs.shape[1]

    def body(outp_ref, hs_ref, dinv_ref, x0_ref, b0_ref, o_ref):
        p = jnp.sum(outp_ref[...], axis=1)
        di = dinv_ref[...]
        o_ref[...] = p + di * di * hs_ref[...] + b0_ref[...] - x0_ref[...]

    return pl.pallas_call(
        body,
        out_shape=jax.ShapeDtypeStruct((NG, n), jnp.float32),
    )(outp, hs, dinv, x0, b0)


def kernel(x, edge_index, edge_attr, batch, W, b):
    n = x.shape[0]
    e = edge_attr.shape[0]
    src = edge_index[0].astype(jnp.int32)
    dst = edge_index[1].astype(jnp.int32)
    w = edge_attr.astype(jnp.float32)

    chunk = NW * L
    e_pad = ((e + chunk - 1) // chunk) * chunk
    if e_pad != e:
        pad = e_pad - e
        src = jnp.concatenate([src, jnp.zeros((pad,), jnp.int32)])
        dst = jnp.concatenate([dst, jnp.zeros((pad,), jnp.int32)])
        w = jnp.concatenate([w, jnp.zeros((pad,), jnp.float32)])

    wc = W[:, :, 0]
    b0 = b[:, 0].reshape(NG, 1)
    x0 = x[:, 0].reshape(1, n)

    hs = _hs_tc(x, wc)
    outp, dinv_rows = _sc_mega_fn(e_pad, n)(
        src, dst, w, hs.reshape(NG * n))
    dinv = dinv_rows[0][:n].reshape(1, n)
    v = _epilogue_tc(outp.reshape(NG, TPG, n), hs, dinv, x0, b0)
    return v.reshape(NG, n // EMBED, EMBED)


# speedup vs baseline: 153.9702x; 1.0779x over previous
import functools

import jax
import jax.numpy as jnp
from jax import lax
from jax.experimental import pallas as pl
from jax.experimental.pallas import tpu as pltpu
from jax.experimental.pallas import tpu_sc as plsc

L = 16
NC = 2
NS = 16
NW = NC * NS
NG = 4
GPC = NG // NC
TPG = NS // GPC
EMBED = 8
RSQRT_MAGIC = 0x5F3759DF


def _sc_mega_fn(e_pad, n):
    ept_d = e_pad // NS
    ept_m = e_pad // TPG
    half = ept_d // 2
    n_pad = -(-n // 2048) * 2048
    npt = n_pad // NS
    mesh = plsc.VectorSubcoreMesh(core_axis_name="c", subcore_axis_name="s")

    @functools.partial(
        pl.kernel,
        mesh=mesh,
        out_type=(
            jax.ShapeDtypeStruct((NW, n), jnp.float32),
            jax.ShapeDtypeStruct((NC, n_pad), jnp.float32),
        ),
        scratch_types=[
            pltpu.VMEM((half,), jnp.int32),
            pltpu.VMEM((half,), jnp.int32),
            pltpu.VMEM((half,), jnp.float32),
            pltpu.VMEM((half,), jnp.int32),
            pltpu.VMEM((half,), jnp.int32),
            pltpu.VMEM((half,), jnp.float32),
            pltpu.VMEM((n_pad,), jnp.float32),
            pltpu.VMEM((2 * npt,), jnp.float32),
            pltpu.VMEM((n,), jnp.float32),
            pltpu.VMEM((n,), jnp.float32),
            pltpu.VMEM_SHARED((NC, NS, n_pad), jnp.float32),
            pltpu.VMEM_SHARED((NC, n_pad), jnp.float32),
            pltpu.SemaphoreType.DMA,
            pltpu.SemaphoreType.DMA,
            pltpu.SemaphoreType.DMA,
        ],
        compiler_params=pltpu.CompilerParams(needs_layout_passes=False),
    )
    def k(src_hbm, dst_hbm, w_hbm, hs_hbm, out_hbm, dinv_hbm,
          src_a, dst_a, w_a, src_b, dst_b, w_b, dd, sl, hs_v, acc,
          shp, shd, sem_a, sem_b, sem_h):
        cid = lax.axis_index("c")
        sid = lax.axis_index("s")
        tid = cid * NS + sid
        gene = cid * GPC + sid // TPG
        base_d = sid * ept_d
        base_m = (sid % TPG) * ept_m

        def fire_deg(d_, w_, sem, r):
            return (
                pltpu.async_copy(
                    dst_hbm.at[pl.ds(base_d + r * half, half)], d_, sem),
                pltpu.async_copy(
                    w_hbm.at[pl.ds(base_d + r * half, half)], w_, sem),
            )

        def fire_msg(s_, d_, w_, sem, r):
            return (
                pltpu.async_copy(
                    src_hbm.at[pl.ds(base_m + r * half, half)], s_, sem),
                pltpu.async_copy(
                    dst_hbm.at[pl.ds(base_m + r * half, half)], d_, sem),
                pltpu.async_copy(
                    w_hbm.at[pl.ds(base_m + r * half, half)], w_, sem),
            )

        cps_a = fire_deg(dst_a, w_a, sem_a, 0)
        cps_b = fire_deg(dst_b, w_b, sem_b, 1)
        cp_h = pltpu.async_copy(hs_hbm.at[pl.ds(gene * n, n)], hs_v, sem_h)

        @plsc.parallel_loop(0, n_pad, L, unroll=8)
        def _(i):
            dd[pl.ds(i, L)] = jnp.zeros((L,), jnp.float32)

        @plsc.parallel_loop(0, n, L, unroll=8)
        def _(i):
            acc[pl.ds(i, L)] = jnp.zeros((L,), jnp.float32)

        def deg_round(d_, w_):
            @plsc.parallel_loop(0, half, L, unroll=4)
            def _(i):
                d = d_[pl.ds(i, L)]
                wv = w_[pl.ds(i, L)]
                plsc.addupdate_scatter(dd, [d], wv)

        for cp in cps_a:
            cp.wait()
        deg_round(dst_a, w_a)
        cps_a = fire_msg(src_a, dst_a, w_a, sem_a, 0)
        for cp in cps_b:
            cp.wait()
        deg_round(dst_b, w_b)
        cps_b = fire_msg(src_b, dst_b, w_b, sem_b, 1)

        pltpu.sync_copy(dd, shp.at[cid, sid])
        plsc.subcore_barrier()
        pltpu.sync_copy(shp.at[cid, 0, pl.ds(sid * npt, npt)],
                        sl.at[pl.ds(0, npt)])
        for t in range(1, NS):
            pltpu.sync_copy(shp.at[cid, t, pl.ds(sid * npt, npt)],
                            sl.at[pl.ds(npt, npt)])

            @plsc.parallel_loop(0, npt, L)
            def _(i):
                sl[pl.ds(i, L)] = sl[pl.ds(i, L)] + sl[pl.ds(npt + i, L)]

        @plsc.parallel_loop(0, npt, L, unroll=2)
        def _(i):
            deg = sl[pl.ds(i, L)] + 1.0
            ii = jnp.int32(RSQRT_MAGIC) - lax.shift_right_logical(
                plsc.bitcast(deg, jnp.int32), 1)
            y = plsc.bitcast(ii, jnp.float32)
            for _ in range(3):
                y = y * (1.5 - 0.5 * deg * y * y)
            sl[pl.ds(i, L)] = jnp.where(deg > 0, y, 0.0)

        pltpu.sync_copy(sl.at[pl.ds(0, npt)], shd.at[cid, pl.ds(sid * npt, npt)])
        plsc.subcore_barrier()
        pltpu.sync_copy(shd.at[cid], dd)

        @pl.when(sid == 0)
        def _():
            pltpu.sync_copy(dd, dinv_hbm.at[cid])

        cp_h.wait()

        @plsc.parallel_loop(0, n, L, unroll=4)
        def _(i):
            hs_v[pl.ds(i, L)] = hs_v[pl.ds(i, L)] * dd[pl.ds(i, L)]

        def msg_round(s_, d_, w_):
            @plsc.parallel_loop(0, half, L, unroll=2)
            def _(i):
                s = s_[pl.ds(i, L)]
                d = d_[pl.ds(i, L)]
                wv = w_[pl.ds(i, L)]
                zs = plsc.load_gather(hs_v, [s])
                di_d = plsc.load_gather(dd, [d])
                plsc.addupdate_scatter(acc, [d], wv * zs * di_d)

        rounds = ept_m // half
        for r in range(rounds):
            if r % 2 == 0:
                for cp in cps_a:
                    cp.wait()
                msg_round(src_a, dst_a, w_a)
                if r + 2 < rounds:
                    cps_a = fire_msg(src_a, dst_a, w_a, sem_a, r + 2)
            else:
                for cp in cps_b:
                    cp.wait()
                msg_round(src_b, dst_b, w_b)
                if r + 2 < rounds:
                    cps_b = fire_msg(src_b, dst_b, w_b, sem_b, r + 2)

        pltpu.sync_copy(acc, out_hbm.at[tid])

    return k


def _hs_tc(x, wc):
    n = x.shape[0]

    def body(x_ref, wc_ref, hs_ref):
        hs_ref[...] = lax.dot_general(
            wc_ref[...], x_ref[...], (((1,), (1,)), ((), ())),
            preferred_element_type=jnp.float32)

    return pl.pallas_call(
        body,
        out_shape=jax.ShapeDtypeStruct((NG, n), jnp.float32),
    )(x, wc)


def _epilogue_tc(outp, hs, dinv, x0, b0):
    n = hs.shape[1]

    def body(outp_ref, hs_ref, dinv_ref, x0_ref, b0_ref, o_ref):
        p = jnp.sum(outp_ref[...], axis=1)
        di = dinv_ref[...]
        o_ref[...] = p + di * di * hs_ref[...] + b0_ref[...] - x0_ref[...]

    return pl.pallas_call(
        body,
        out_shape=jax.ShapeDtypeStruct((NG, n), jnp.float32),
    )(outp, hs, dinv, x0, b0)


def kernel(x, edge_index, edge_attr, batch, W, b):
    n = x.shape[0]
    e = edge_attr.shape[0]
    src = edge_index[0].astype(jnp.int32)
    dst = edge_index[1].astype(jnp.int32)
    w = edge_attr.astype(jnp.float32)

    chunk = NW * L
    e_pad = ((e + chunk - 1) // chunk) * chunk
    if e_pad != e:
        pad = e_pad - e
        src = jnp.concatenate([src, jnp.zeros((pad,), jnp.int32)])
        dst = jnp.concatenate([dst, jnp.zeros((pad,), jnp.int32)])
        w = jnp.concatenate([w, jnp.zeros((pad,), jnp.float32)])

    wc = W[:, :, 0]
    b0 = b[:, 0].reshape(NG, 1)
    x0 = x[:, 0].reshape(1, n)

    hs = _hs_tc(x, wc)
    outp, dinv_rows = _sc_mega_fn(e_pad, n)(
        src, dst, w, hs.reshape(NG * n))
    dinv = dinv_rows[0][:n].reshape(1, n)
    v = _epilogue_tc(outp.reshape(NG, TPG, n), hs, dinv, x0, b0)
    return v.reshape(NG, n // EMBED, EMBED)


# speedup vs baseline: 165.4452x; 1.0745x over previous
import functools

import jax
import jax.numpy as jnp
from jax import lax
from jax.experimental import pallas as pl
from jax.experimental.pallas import tpu as pltpu
from jax.experimental.pallas import tpu_sc as plsc

L = 16
NC = 2
NS = 16
NW = NC * NS
NG = 4
GPC = NG // NC
TPG = NS // GPC
EMBED = 8
RSQRT_MAGIC = 0x5F3759DF


def _sc_mega_fn(e, n):
    ept_d = e // NS
    ept_m = e // TPG
    half = ept_d // 2
    full = half // L * L
    rem = half - full
    buflen = -(-half // L) * L
    n_pad = -(-n // 2048) * 2048
    npt = n_pad // NS
    mesh = plsc.VectorSubcoreMesh(core_axis_name="c", subcore_axis_name="s")

    @functools.partial(
        pl.kernel,
        mesh=mesh,
        out_type=(
            jax.ShapeDtypeStruct((NW, n), jnp.float32),
            jax.ShapeDtypeStruct((NC, n_pad), jnp.float32),
        ),
        scratch_types=[
            pltpu.VMEM((buflen,), jnp.int32),
            pltpu.VMEM((buflen,), jnp.int32),
            pltpu.VMEM((buflen,), jnp.float32),
            pltpu.VMEM((buflen,), jnp.int32),
            pltpu.VMEM((buflen,), jnp.int32),
            pltpu.VMEM((buflen,), jnp.float32),
            pltpu.VMEM((n_pad,), jnp.float32),
            pltpu.VMEM((2 * npt,), jnp.float32),
            pltpu.VMEM((n,), jnp.float32),
            pltpu.VMEM((n,), jnp.float32),
            pltpu.VMEM_SHARED((NC, NS, n_pad), jnp.float32),
            pltpu.VMEM_SHARED((NC, n_pad), jnp.float32),
            pltpu.SemaphoreType.DMA,
            pltpu.SemaphoreType.DMA,
            pltpu.SemaphoreType.DMA,
        ],
        compiler_params=pltpu.CompilerParams(needs_layout_passes=False),
    )
    def k(src_hbm, dst_hbm, w_hbm, hs_hbm, out_hbm, dinv_hbm,
          src_a, dst_a, w_a, src_b, dst_b, w_b, dd, sl, hs_v, acc,
          shp, shd, sem_a, sem_b, sem_h):
        cid = lax.axis_index("c")
        sid = lax.axis_index("s")
        tid = cid * NS + sid
        gene = cid * GPC + sid // TPG
        base_d = sid * ept_d
        base_m = (sid % TPG) * ept_m

        def fire_deg(d_, w_, sem, r):
            return (
                pltpu.async_copy(
                    dst_hbm.at[pl.ds(base_d + r * half, half)],
                    d_.at[pl.ds(0, half)], sem),
                pltpu.async_copy(
                    w_hbm.at[pl.ds(base_d + r * half, half)],
                    w_.at[pl.ds(0, half)], sem),
            )

        def fire_msg(s_, d_, w_, sem, r):
            return (
                pltpu.async_copy(
                    src_hbm.at[pl.ds(base_m + r * half, half)],
                    s_.at[pl.ds(0, half)], sem),
                pltpu.async_copy(
                    dst_hbm.at[pl.ds(base_m + r * half, half)],
                    d_.at[pl.ds(0, half)], sem),
                pltpu.async_copy(
                    w_hbm.at[pl.ds(base_m + r * half, half)],
                    w_.at[pl.ds(0, half)], sem),
            )

        cps_a = fire_deg(dst_a, w_a, sem_a, 0)
        cps_b = fire_deg(dst_b, w_b, sem_b, 1)
        cp_h = pltpu.async_copy(hs_hbm.at[gene], hs_v, sem_h)

        @plsc.parallel_loop(0, n_pad, L, unroll=8)
        def _(i):
            dd[pl.ds(i, L)] = jnp.zeros((L,), jnp.float32)

        @plsc.parallel_loop(0, n, L, unroll=8)
        def _(i):
            acc[pl.ds(i, L)] = jnp.zeros((L,), jnp.float32)

        def deg_round(d_, w_):
            @plsc.parallel_loop(0, full, L, unroll=4)
            def _(i):
                d = d_[pl.ds(i, L)]
                wv = w_[pl.ds(i, L)]
                plsc.addupdate_scatter(dd, [d], wv)

            if rem:
                m = lax.iota(jnp.int32, L) < rem
                d = jnp.where(m, d_[pl.ds(full, L)], 0)
                wv = jnp.where(m, w_[pl.ds(full, L)], 0.0)
                plsc.addupdate_scatter(dd, [d], wv)

        for cp in cps_a:
            cp.wait()
        deg_round(dst_a, w_a)
        cps_a = fire_msg(src_a, dst_a, w_a, sem_a, 0)
        for cp in cps_b:
            cp.wait()
        deg_round(dst_b, w_b)
        cps_b = fire_msg(src_b, dst_b, w_b, sem_b, 1)

        pltpu.sync_copy(dd, shp.at[cid, sid])
        plsc.subcore_barrier()
        pltpu.sync_copy(shp.at[cid, 0, pl.ds(sid * npt, npt)],
                        sl.at[pl.ds(0, npt)])
        for t in range(1, NS):
            pltpu.sync_copy(shp.at[cid, t, pl.ds(sid * npt, npt)],
                            sl.at[pl.ds(npt, npt)])

            @plsc.parallel_loop(0, npt, L)
            def _(i):
                sl[pl.ds(i, L)] = sl[pl.ds(i, L)] + sl[pl.ds(npt + i, L)]

        @plsc.parallel_loop(0, npt, L, unroll=2)
        def _(i):
            deg = sl[pl.ds(i, L)] + 1.0
            ii = jnp.int32(RSQRT_MAGIC) - lax.shift_right_logical(
                plsc.bitcast(deg, jnp.int32), 1)
            y = plsc.bitcast(ii, jnp.float32)
            for _ in range(3):
                y = y * (1.5 - 0.5 * deg * y * y)
            sl[pl.ds(i, L)] = jnp.where(deg > 0, y, 0.0)

        pltpu.sync_copy(sl.at[pl.ds(0, npt)], shd.at[cid, pl.ds(sid * npt, npt)])
        plsc.subcore_barrier()
        pltpu.sync_copy(shd.at[cid], dd)

        @pl.when(sid == 0)
        def _():
            pltpu.sync_copy(dd, dinv_hbm.at[cid])

        cp_h.wait()

        @plsc.parallel_loop(0, n, L, unroll=4)
        def _(i):
            hs_v[pl.ds(i, L)] = hs_v[pl.ds(i, L)] * dd[pl.ds(i, L)]

        def msg_round(s_, d_, w_):
            @plsc.parallel_loop(0, full, L, unroll=2)
            def _(i):
                s = s_[pl.ds(i, L)]
                d = d_[pl.ds(i, L)]
                wv = w_[pl.ds(i, L)]
                zs = plsc.load_gather(hs_v, [s])
                di_d = plsc.load_gather(dd, [d])
                plsc.addupdate_scatter(acc, [d], wv * zs * di_d)

            if rem:
                m = lax.iota(jnp.int32, L) < rem
                s = jnp.where(m, s_[pl.ds(full, L)], 0)
                d = jnp.where(m, d_[pl.ds(full, L)], 0)
                wv = jnp.where(m, w_[pl.ds(full, L)], 0.0)
                zs = plsc.load_gather(hs_v, [s])
                di_d = plsc.load_gather(dd, [d])
                plsc.addupdate_scatter(acc, [d], wv * zs * di_d)

        rounds = ept_m // half
        for r in range(rounds):
            if r % 2 == 0:
                for cp in cps_a:
                    cp.wait()
                msg_round(src_a, dst_a, w_a)
                if r + 2 < rounds:
                    cps_a = fire_msg(src_a, dst_a, w_a, sem_a, r + 2)
            else:
                for cp in cps_b:
                    cp.wait()
                msg_round(src_b, dst_b, w_b)
                if r + 2 < rounds:
                    cps_b = fire_msg(src_b, dst_b, w_b, sem_b, r + 2)

        pltpu.sync_copy(acc, out_hbm.at[tid])

    return k


def _hs_tc(x, wc_ext):
    n = x.shape[0]

    def body(x_ref, wc_ref, hs_ref):
        hs_ref[...] = lax.dot_general(
            wc_ref[...], x_ref[...], (((1,), (1,)), ((), ())),
            preferred_element_type=jnp.float32)

    return pl.pallas_call(
        body,
        out_shape=jax.ShapeDtypeStruct((NG + 1, n), jnp.float32),
    )(x, wc_ext)


def _epilogue_tc(outp, hs2, dinvp, b0):
    n = outp.shape[2]

    def body(outp_ref, hs2_ref, dinvp_ref, b0_ref, o_ref):
        p = jnp.sum(outp_ref[...], axis=1)
        hs = hs2_ref[0:NG, :]
        x0 = hs2_ref[NG:NG + 1, :]
        di = dinvp_ref[0:1, 0:n]
        o_ref[...] = p + di * di * hs + b0_ref[...] - x0

    return pl.pallas_call(
        body,
        out_shape=jax.ShapeDtypeStruct((NG, n), jnp.float32),
    )(outp, hs2, dinvp, b0)


def kernel(x, edge_index, edge_attr, batch, W, b):
    n = x.shape[0]
    e = edge_attr.shape[0]
    src = edge_index[0].astype(jnp.int32)
    dst = edge_index[1].astype(jnp.int32)
    w = edge_attr.astype(jnp.float32)

    if e % 256:
        pad = 256 - e % 256
        src = jnp.concatenate([src, jnp.zeros((pad,), jnp.int32)])
        dst = jnp.concatenate([dst, jnp.zeros((pad,), jnp.int32)])
        w = jnp.concatenate([w, jnp.zeros((pad,), jnp.float32)])
        e += pad

    e0 = jnp.zeros((1, x.shape[1]), jnp.float32).at[0, 0].set(1.0)
    wc_ext = jnp.concatenate([W[:, :, 0], e0], axis=0)
    b0 = b[:, 0].reshape(NG, 1)

    hs2 = _hs_tc(x, wc_ext)
    outp, dinvp = _sc_mega_fn(e, n)(src, dst, w, hs2)
    v = _epilogue_tc(outp.reshape(NG, TPG, n), hs2, dinvp, b0)
    return v.reshape(NG, n // EMBED, EMBED)
